# Initial kernel scaffold; baseline (speedup 1.0000x reference)
import jax
import jax.numpy as jnp
from jax.experimental import pallas as pl


def kernel(x, edge_index, W1, b1, W2, b2):
    raise NotImplementedError("write your pallas kernel here")


# speedup vs baseline: 3.7543x; 3.7543x over previous
import functools

import jax
import jax.numpy as jnp
from jax import lax
from jax.experimental import pallas as pl
from jax.experimental.pallas import tpu as pltpu
from jax.experimental.pallas import tpu_sc as plsc

_NUM_CORES = 2
_NUM_SUBCORES = 16


def _largest_chunk(edges_per_worker):
    for ch in range(128, 0, -8):
        if edges_per_worker % ch == 0:
            return ch
    raise ValueError(edges_per_worker)


def _make_sc_agg_edge_split(n, d, e):
    eps = e // (_NUM_CORES * _NUM_SUBCORES)
    assert eps * _NUM_CORES * _NUM_SUBCORES == e
    ch = _largest_chunk(eps)
    nch = eps // ch
    n_pad = -(-n // (8 * _NUM_SUBCORES)) * 8 * _NUM_SUBCORES
    rps = n_pad // _NUM_SUBCORES
    mesh = plsc.VectorSubcoreMesh(core_axis_name="c", subcore_axis_name="s")

    @functools.partial(
        pl.kernel,
        out_type=[jax.ShapeDtypeStruct((n_pad, d), jnp.float32),
                  jax.ShapeDtypeStruct((n_pad, d), jnp.float32)],
        mesh=mesh,
        scratch_types=[
            pltpu.VMEM_SHARED((n_pad, d), jnp.float32),
            pltpu.VMEM((1, ch), jnp.int32),
            pltpu.VMEM((1, ch), jnp.int32),
            pltpu.VMEM((ch, d), jnp.float32),
        ],
    )
    def k(feat_hbm, zeros_hbm, src_hbm, dst_hbm, out0_hbm, out1_hbm,
          acc, sidx, didx, rows):
        cid = lax.axis_index("c")
        sid = lax.axis_index("s")
        pltpu.sync_copy(zeros_hbm.at[pl.ds(sid * rps, rps)],
                        acc.at[pl.ds(sid * rps, rps)])
        plsc.subcore_barrier()

        base = (cid * _NUM_SUBCORES + sid) * eps

        @pl.loop(0, nch)
        def _(j):
            off = pl.multiple_of(base + j * ch, 8)
            pltpu.sync_copy(src_hbm.at[pl.ds(off, ch)], sidx.at[0])
            pltpu.sync_copy(dst_hbm.at[pl.ds(off, ch)], didx.at[0])
            pltpu.sync_copy(feat_hbm.at[sidx.at[0]], rows)
            pltpu.sync_copy(rows, acc.at[didx.at[0]], add=True)

        plsc.subcore_barrier()

        @pl.when(cid == 0)
        def _():
            pltpu.sync_copy(acc.at[pl.ds(sid * rps, rps)],
                            out0_hbm.at[pl.ds(sid * rps, rps)])

        @pl.when(cid == 1)
        def _():
            pltpu.sync_copy(acc.at[pl.ds(sid * rps, rps)],
                            out1_hbm.at[pl.ds(sid * rps, rps)])

    return k


def _make_sc_agg_col_split(n, dh, e):
    eps = e // _NUM_SUBCORES
    assert eps * _NUM_SUBCORES == e
    ch = _largest_chunk(eps)
    nch = eps // ch
    n_pad = -(-n // (8 * _NUM_SUBCORES)) * 8 * _NUM_SUBCORES
    rps = n_pad // _NUM_SUBCORES
    mesh = plsc.VectorSubcoreMesh(core_axis_name="c", subcore_axis_name="s")

    @functools.partial(
        pl.kernel,
        out_type=[jax.ShapeDtypeStruct((n_pad, dh), jnp.float32),
                  jax.ShapeDtypeStruct((n_pad, dh), jnp.float32)],
        mesh=mesh,
        scratch_types=[
            pltpu.VMEM_SHARED((n_pad, dh), jnp.float32),
            pltpu.VMEM((1, ch), jnp.int32),
            pltpu.VMEM((1, ch), jnp.int32),
            pltpu.VMEM((ch, dh), jnp.float32),
        ],
    )
    def k(feat0_hbm, feat1_hbm, zeros_hbm, src_hbm, dst_hbm,
          out0_hbm, out1_hbm, acc, sidx, didx, rows):
        cid = lax.axis_index("c")
        sid = lax.axis_index("s")
        pltpu.sync_copy(zeros_hbm.at[pl.ds(sid * rps, rps)],
                        acc.at[pl.ds(sid * rps, rps)])
        plsc.subcore_barrier()

        base = sid * eps

        def edge_loop(feat_hbm):
            @pl.loop(0, nch)
            def _(j):
                off = pl.multiple_of(base + j * ch, 8)
                pltpu.sync_copy(src_hbm.at[pl.ds(off, ch)], sidx.at[0])
                pltpu.sync_copy(dst_hbm.at[pl.ds(off, ch)], didx.at[0])
                pltpu.sync_copy(feat_hbm.at[sidx.at[0]], rows)
                pltpu.sync_copy(rows, acc.at[didx.at[0]], add=True)

        @pl.when(cid == 0)
        def _():
            edge_loop(feat0_hbm)

        @pl.when(cid == 1)
        def _():
            edge_loop(feat1_hbm)

        plsc.subcore_barrier()

        @pl.when(cid == 0)
        def _():
            pltpu.sync_copy(acc.at[pl.ds(sid * rps, rps)],
                            out0_hbm.at[pl.ds(sid * rps, rps)])

        @pl.when(cid == 1)
        def _():
            pltpu.sync_copy(acc.at[pl.ds(sid * rps, rps)],
                            out1_hbm.at[pl.ds(sid * rps, rps)])

    return k


def _tc_layer1(x, p0, p1, w, b):
    n, d_in = x.shape
    d_out = w.shape[1]
    dh = d_out // 2
    br = 1000
    grid = (n // br,)

    def body(x_ref, p0_ref, p1_ref, w_ref, b_ref, o0_ref, o1_ref):
        h = x_ref[...] + p0_ref[...] + p1_ref[...]
        y = lax.dot_general(h, w_ref[...], (((1,), (0,)), ((), ())),
                            precision=lax.Precision.HIGHEST,
                            preferred_element_type=jnp.float32)
        y = jnp.maximum(y + b_ref[...], 0.0)
        o0_ref[...] = y[:, :dh]
        o1_ref[...] = y[:, dh:]

    return pl.pallas_call(
        body,
        grid=grid,
        in_specs=[
            pl.BlockSpec((br, d_in), lambda i: (i, 0)),
            pl.BlockSpec((br, d_in), lambda i: (i, 0)),
            pl.BlockSpec((br, d_in), lambda i: (i, 0)),
            pl.BlockSpec((d_in, d_out), lambda i: (0, 0)),
            pl.BlockSpec((1, d_out), lambda i: (0, 0)),
        ],
        out_specs=[
            pl.BlockSpec((br, dh), lambda i: (i, 0)),
            pl.BlockSpec((br, dh), lambda i: (i, 0)),
        ],
        out_shape=[jax.ShapeDtypeStruct((n, dh), jnp.float32),
                   jax.ShapeDtypeStruct((n, dh), jnp.float32)],
    )(x, p0, p1, w, b.reshape(1, d_out))


def _tc_layer2(h0, h1, a0, a1, w, b):
    n, dh = h0.shape
    d_out = w.shape[1]
    br = 1000
    grid = (n // br,)

    def body(h0_ref, h1_ref, a0_ref, a1_ref, w_ref, b_ref, o_ref):
        h = jnp.concatenate([h0_ref[...] + a0_ref[...],
                             h1_ref[...] + a1_ref[...]], axis=1)
        y = lax.dot_general(h, w_ref[...], (((1,), (0,)), ((), ())),
                            precision=lax.Precision.HIGHEST,
                            preferred_element_type=jnp.float32)
        o_ref[...] = y + b_ref[...]

    return pl.pallas_call(
        body,
        grid=grid,
        in_specs=[
            pl.BlockSpec((br, dh), lambda i: (i, 0)),
            pl.BlockSpec((br, dh), lambda i: (i, 0)),
            pl.BlockSpec((br, dh), lambda i: (i, 0)),
            pl.BlockSpec((br, dh), lambda i: (i, 0)),
            pl.BlockSpec((2 * dh, d_out), lambda i: (0, 0)),
            pl.BlockSpec((1, d_out), lambda i: (0, 0)),
        ],
        out_specs=pl.BlockSpec((br, d_out), lambda i: (i, 0)),
        out_shape=jax.ShapeDtypeStruct((n, d_out), jnp.float32),
    )(h0, h1, a0, a1, w, b.reshape(1, d_out))


def kernel(x, edge_index, W1, b1, W2, b2):
    n, d_in = x.shape
    e = edge_index.shape[1]
    d_hid = W1.shape[1]
    dh = d_hid // 2

    src = edge_index[0].astype(jnp.int32)
    dst = edge_index[1].astype(jnp.int32)

    n_pad = -(-n // (8 * _NUM_SUBCORES)) * 8 * _NUM_SUBCORES
    zeros_full = jnp.zeros((n_pad, d_in), jnp.float32)
    zeros_half = jnp.zeros((n_pad, dh), jnp.float32)

    sc1 = _make_sc_agg_edge_split(n, d_in, e)
    p0, p1 = sc1(x, zeros_full, src, dst)
    h0, h1 = _tc_layer1(x, p0, p1, W1, b1)

    sc2 = _make_sc_agg_col_split(n, dh, e)
    a0, a1 = sc2(h0, h1, zeros_half, src, dst)
    return _tc_layer2(h0, h1, a0, a1, W2, b2)
